# Initial kernel scaffold; baseline (speedup 1.0000x reference)
import jax
import jax.numpy as jnp
from jax.experimental import pallas as pl


def kernel(field, price, rating, brand, name, category, description, emb_table, W_price, W_rating, W_fc, b_fc, W_lin):
    raise NotImplementedError("write your pallas kernel here")


# speedup vs baseline: 1.6665x; 1.6665x over previous
import functools

import jax
import jax.numpy as jnp
from jax import lax
from jax.experimental import pallas as pl
from jax.experimental.pallas import tpu as pltpu
import jax.experimental.pallas.tpu_sc as plsc

_D = 1024
_V = 50265
_B = 1024

_NC = 2
_NS = 16
_NW = _NC * _NS
_BPW = _B // _NW
_SEGL = (16, 32, 96, 96, 96)
_SEGO = (0, 16, 48, 144, 240)
_TPB = 336

_GROWS = 512
_GBLK = (_V + _GROWS - 1) // _GROWS


def _np_body(fnp_ref, wfc1_ref, out_ref):
    out_ref[...] = jnp.dot(fnp_ref[...], wfc1_ref[...],
                           preferred_element_type=jnp.float32)


def _gtab_body(x_ref, wfc2_ref, wlin_ref, np_ref, bfc_ref, out_ref):
    h = jnp.dot(x_ref[...], wfc2_ref[...], preferred_element_type=jnp.float32)
    h = jnp.maximum(h + np_ref[5:6, :] + bfc_ref[...], 0.0)
    out_ref[...] = jnp.dot(h, wlin_ref[...], preferred_element_type=jnp.float32)


def _small_body(pp_ref, rp_ref, wp_ref, wr_ref, s_ref, w_ref, np_ref,
                bfc_ref, wfc2_ref, wlin_ref, out_ref):
    pv = jnp.dot(pp_ref[...], wp_ref[...], preferred_element_type=jnp.float32)
    rv = jnp.dot(rp_ref[...], wr_ref[...], preferred_element_type=jnp.float32)
    bs = s_ref[:, 0, :]
    ns = s_ref[:, 1, :]
    cv = (w_ref[:, 0:1] * s_ref[:, 2, :] + w_ref[:, 1:2] * s_ref[:, 3, :]
          + w_ref[:, 2:3] * s_ref[:, 4, :])
    for t, v in enumerate((pv, rv, bs, ns, cv)):
        h = jnp.dot(v, wfc2_ref[...], preferred_element_type=jnp.float32)
        h = jnp.maximum(h + np_ref[t:t + 1, :] + bfc_ref[...], 0.0)
        out_ref[:, t, :] = jnp.dot(h, wlin_ref[...],
                                   preferred_element_type=jnp.float32)


def _seg_body(emb_hbm, tok_hbm, init_hbm, out_hbm, idx_v, buf_v, acc_v, sem):
    wid = lax.axis_index("s") * _NC + lax.axis_index("c")

    def b_body(j, carry):
        b = wid * _BPW + j
        pltpu.sync_copy(tok_hbm.at[pl.ds(b * _TPB, _TPB)], idx_v)
        pltpu.sync_copy(init_hbm.at[pl.ds(b * 5, 5)], acc_v)
        for s in range(5):
            n = _SEGL[s]
            pltpu.async_copy(emb_hbm.at[idx_v.at[pl.ds(_SEGO[s], n)]],
                             buf_v.at[pl.ds(0, n)], sem).wait()

            def col_body(col, c2):
                co = pl.multiple_of(col * 16, 16)

                def r_body(r, acc):
                    return acc + buf_v[r, pl.ds(co, 16)]

                acc = lax.fori_loop(0, n, r_body, acc_v[s, pl.ds(co, 16)])
                acc_v[s, pl.ds(co, 16)] = acc
                return c2

            lax.fori_loop(0, _D // 16, col_body, 0)
        pltpu.sync_copy(acc_v, out_hbm.at[pl.ds(b * 5, 5)])
        return carry

    lax.fori_loop(0, _BPW, b_body, 0)


def _asm_body(g_hbm, small_hbm, didx_hbm, out_hbm, idx_v, rows_v, small_v, sem):
    wid = lax.axis_index("s") * _NC + lax.axis_index("c")

    def b_body(j, carry):
        b = wid * _BPW + j
        pltpu.sync_copy(small_hbm.at[pl.ds(b * 5, 5)], small_v)
        pltpu.sync_copy(small_v, out_hbm.at[pl.ds(b * 133, 5)])

        def c_body(ci, c2):
            pltpu.sync_copy(didx_hbm.at[pl.ds(b * 128 + ci * 64, 64)], idx_v)
            pltpu.async_copy(g_hbm.at[idx_v], rows_v, sem).wait()
            pltpu.sync_copy(rows_v,
                            out_hbm.at[pl.ds(b * 133 + 5 + ci * 64, 64)])
            return c2

        lax.fori_loop(0, 2, c_body, 0)
        return carry

    lax.fori_loop(0, _BPW, b_body, 0)


def kernel(field, price, rating, brand, name, category, description,
           emb_table, W_price, W_rating, W_fc, b_fc, W_lin):
    f32 = jnp.float32
    i32 = jnp.int32
    wfc1 = W_fc[:_D]
    wfc2 = W_fc[_D:]
    bfc2d = b_fc.reshape(1, _D)

    fnp = jnp.take(emb_table, field[:, 0], axis=0)
    fnp = jnp.concatenate([fnp, jnp.zeros((2, _D), f32)], axis=0)
    nprj = pl.pallas_call(
        _np_body,
        out_shape=jax.ShapeDtypeStruct((8, _D), f32),
    )(fnp, wfc1)

    gtab = pl.pallas_call(
        _gtab_body,
        grid=(_GBLK,),
        in_specs=[
            pl.BlockSpec((_GROWS, _D), lambda i: (i, 0)),
            pl.BlockSpec((_D, _D), lambda i: (0, 0)),
            pl.BlockSpec((_D, _D), lambda i: (0, 0)),
            pl.BlockSpec((8, _D), lambda i: (0, 0)),
            pl.BlockSpec((1, _D), lambda i: (0, 0)),
        ],
        out_specs=pl.BlockSpec((_GROWS, _D), lambda i: (i, 0)),
        out_shape=jax.ShapeDtypeStruct((_V, _D), f32),
    )(emb_table, wfc2, W_lin, nprj, bfc2d)

    cat3 = category.reshape(_B, 3, 96).astype(i32)
    tokp = jnp.concatenate([
        jnp.pad(brand.astype(i32), ((0, 0), (0, 4)), constant_values=1),
        name.astype(i32),
        cat3.reshape(_B, 288),
    ], axis=1)
    seg_id = jnp.repeat(jnp.arange(5, dtype=i32),
                        jnp.asarray(_SEGL), total_repeat_length=_TPB)
    cnt1 = jnp.zeros((_B, 5), f32).at[
        jnp.arange(_B)[:, None], seg_id[None, :]].add(
        (tokp == 1).astype(f32))
    init = (-cnt1.reshape(-1)[:, None]) * emb_table[1][None, :]

    mesh = plsc.VectorSubcoreMesh(core_axis_name="c", subcore_axis_name="s")
    segsum = pl.kernel(
        _seg_body,
        mesh=mesh,
        compiler_params=pltpu.CompilerParams(use_tc_tiling_on_sc=False),
        out_type=jax.ShapeDtypeStruct((_B * 5, _D), f32),
        scratch_types=[
            pltpu.VMEM((_TPB,), i32),
            pltpu.VMEM((96, _D), f32),
            pltpu.VMEM((5, _D), f32),
            pltpu.SemaphoreType.DMA,
        ],
    )(emb_table, tokp.reshape(-1), init)

    cmask = category != 1
    any1 = jnp.any(cmask, axis=-1)
    n1 = any1.sum(-1).astype(f32)
    n2 = jnp.any(any1, axis=-1).sum(-1).astype(f32)
    w3 = 1.0 / (n1 + 1e-6) / (n2 + 1e-6)[:, None]
    wpad = jnp.pad(w3, ((0, 0), (0, 125)))

    ppad = jnp.pad(price.astype(f32), ((0, 0), (0, 128 - 11)))
    rpad = jnp.pad(rating.astype(f32), ((0, 0), (0, 128 - 4)))
    wppad = jnp.pad(W_price, ((0, 128 - 11), (0, 0)))
    wrpad = jnp.pad(W_rating, ((0, 128 - 4), (0, 0)))
    bb = 128
    small = pl.pallas_call(
        _small_body,
        grid=(_B // bb,),
        in_specs=[
            pl.BlockSpec((bb, 128), lambda i: (i, 0)),
            pl.BlockSpec((bb, 128), lambda i: (i, 0)),
            pl.BlockSpec((128, _D), lambda i: (0, 0)),
            pl.BlockSpec((128, _D), lambda i: (0, 0)),
            pl.BlockSpec((bb, 5, _D), lambda i: (i, 0, 0)),
            pl.BlockSpec((bb, 128), lambda i: (i, 0)),
            pl.BlockSpec((8, _D), lambda i: (0, 0)),
            pl.BlockSpec((1, _D), lambda i: (0, 0)),
            pl.BlockSpec((_D, _D), lambda i: (0, 0)),
            pl.BlockSpec((_D, _D), lambda i: (0, 0)),
        ],
        out_specs=pl.BlockSpec((bb, 5, _D), lambda i: (i, 0, 0)),
        out_shape=jax.ShapeDtypeStruct((_B, 5, _D), f32),
    )(ppad, rpad, wppad, wrpad, segsum.reshape(_B, 5, _D), wpad,
      nprj, bfc2d, wfc2, W_lin)


    out_flat = pl.kernel(
        _asm_body,
        mesh=mesh,
        compiler_params=pltpu.CompilerParams(use_tc_tiling_on_sc=False),
        out_type=jax.ShapeDtypeStruct((_B * 133, _D), f32),
        scratch_types=[
            pltpu.VMEM((64,), i32),
            pltpu.VMEM((64, _D), f32),
            pltpu.VMEM((5, _D), f32),
            pltpu.SemaphoreType.DMA,
        ],
    )(gtab, small.reshape(_B * 5, _D), description.reshape(-1).astype(i32))

    all_embeddings = out_flat.reshape(_B, 133, _D)
    all_masks = jnp.concatenate([
        price.sum(axis=1, keepdims=True) != 0.0,
        jnp.ones((_B, 1), bool),
        brand[:, :1] != 1,
        name[:, :1] != 1,
        jnp.ones((_B, 1), bool),
        description != 1,
    ], axis=1)
    return all_embeddings, all_masks


# speedup vs baseline: 2.7003x; 1.6204x over previous
import functools

import jax
import jax.numpy as jnp
from jax import lax
from jax.experimental import pallas as pl
from jax.experimental.pallas import tpu as pltpu
import jax.experimental.pallas.tpu_sc as plsc

_D = 1024
_V = 50265
_B = 1024

_NC = 2
_NS = 16
_NW = _NC * _NS
_BPW = _B // _NW
_SEGL = (16, 32, 96, 96, 96)
_SEGO = (0, 16, 48, 144, 240)
_TPB = 336

_GROWS = 512
_GBLK = (_V + _GROWS - 1) // _GROWS


def _np_body(fnp_ref, wfc1_ref, out_ref):
    out_ref[...] = jnp.dot(fnp_ref[...], wfc1_ref[...],
                           preferred_element_type=jnp.float32)


def _gtab_body(x_ref, wfc2_ref, wlin_ref, np_ref, bfc_ref, out_ref):
    h = jnp.dot(x_ref[...], wfc2_ref[...], preferred_element_type=jnp.float32)
    h = jnp.maximum(h + np_ref[5:6, :] + bfc_ref[...], 0.0)
    out_ref[...] = jnp.dot(h, wlin_ref[...], preferred_element_type=jnp.float32)


def _small_body(pp_ref, rp_ref, wp_ref, wr_ref, s_ref, w_ref, np_ref,
                bfc_ref, wfc2_ref, wlin_ref, out_ref):
    pv = jnp.dot(pp_ref[...], wp_ref[...], preferred_element_type=jnp.float32)
    rv = jnp.dot(rp_ref[...], wr_ref[...], preferred_element_type=jnp.float32)
    bs = s_ref[:, 0, :]
    ns = s_ref[:, 1, :]
    cv = (w_ref[:, 0:1] * s_ref[:, 2, :] + w_ref[:, 1:2] * s_ref[:, 3, :]
          + w_ref[:, 2:3] * s_ref[:, 4, :])
    for t, v in enumerate((pv, rv, bs, ns, cv)):
        h = jnp.dot(v, wfc2_ref[...], preferred_element_type=jnp.float32)
        h = jnp.maximum(h + np_ref[t:t + 1, :] + bfc_ref[...], 0.0)
        out_ref[:, t, :] = jnp.dot(h, wlin_ref[...],
                                   preferred_element_type=jnp.float32)


def _seg_body(emb_hbm, tok_hbm, init_hbm, out_hbm, idx_v, buf_v, acc_v, sem):
    wid = lax.axis_index("s") * _NC + lax.axis_index("c")

    def b_body(j, carry):
        b = wid * _BPW + j
        pltpu.sync_copy(tok_hbm.at[pl.ds(b * _TPB, _TPB)], idx_v)
        pltpu.sync_copy(init_hbm.at[pl.ds(b * 5, 5)], acc_v)
        for s in range(5):
            n = _SEGL[s]
            pltpu.async_copy(emb_hbm.at[idx_v.at[pl.ds(_SEGO[s], n)]],
                             buf_v.at[pl.ds(0, n)], sem).wait()

            def cb_body(cb, c2):
                base = pl.multiple_of(cb * 256, 256)

                def r_body(r, accs):
                    return tuple(accs[k] + buf_v[r, pl.ds(base + k * 16, 16)]
                                 for k in range(16))

                acc0 = tuple(acc_v[s, pl.ds(base + k * 16, 16)]
                             for k in range(16))
                accs = lax.fori_loop(0, n, r_body, acc0)
                for k in range(16):
                    acc_v[s, pl.ds(base + k * 16, 16)] = accs[k]
                return c2

            lax.fori_loop(0, _D // 256, cb_body, 0)
        pltpu.sync_copy(acc_v, out_hbm.at[pl.ds(b * 5, 5)])
        return carry

    lax.fori_loop(0, _BPW, b_body, 0)


def _asm_body(g_hbm, small_hbm, didx_hbm, out_hbm, idx_v, rows_v, small_v, sem):
    wid = lax.axis_index("s") * _NC + lax.axis_index("c")

    def b_body(j, carry):
        b = wid * _BPW + j
        pltpu.sync_copy(small_hbm.at[pl.ds(b * 5, 5)], small_v)
        pltpu.sync_copy(small_v, out_hbm.at[pl.ds(b * 133, 5)])

        def c_body(ci, c2):
            pltpu.sync_copy(didx_hbm.at[pl.ds(b * 128 + ci * 64, 64)], idx_v)
            pltpu.async_copy(g_hbm.at[idx_v], rows_v, sem).wait()
            pltpu.sync_copy(rows_v,
                            out_hbm.at[pl.ds(b * 133 + 5 + ci * 64, 64)])
            return c2

        lax.fori_loop(0, 2, c_body, 0)
        return carry

    lax.fori_loop(0, _BPW, b_body, 0)


def kernel(field, price, rating, brand, name, category, description,
           emb_table, W_price, W_rating, W_fc, b_fc, W_lin):
    f32 = jnp.float32
    i32 = jnp.int32
    wfc1 = W_fc[:_D]
    wfc2 = W_fc[_D:]
    bfc2d = b_fc.reshape(1, _D)

    fnp = jnp.take(emb_table, field[:, 0], axis=0)
    fnp = jnp.concatenate([fnp, jnp.zeros((2, _D), f32)], axis=0)
    nprj = pl.pallas_call(
        _np_body,
        out_shape=jax.ShapeDtypeStruct((8, _D), f32),
    )(fnp, wfc1)

    gtab = pl.pallas_call(
        _gtab_body,
        grid=(_GBLK,),
        in_specs=[
            pl.BlockSpec((_GROWS, _D), lambda i: (i, 0)),
            pl.BlockSpec((_D, _D), lambda i: (0, 0)),
            pl.BlockSpec((_D, _D), lambda i: (0, 0)),
            pl.BlockSpec((8, _D), lambda i: (0, 0)),
            pl.BlockSpec((1, _D), lambda i: (0, 0)),
        ],
        out_specs=pl.BlockSpec((_GROWS, _D), lambda i: (i, 0)),
        out_shape=jax.ShapeDtypeStruct((_V, _D), f32),
    )(emb_table, wfc2, W_lin, nprj, bfc2d)

    cat3 = category.reshape(_B, 3, 96).astype(i32)
    tokp = jnp.concatenate([
        jnp.pad(brand.astype(i32), ((0, 0), (0, 4)), constant_values=1),
        name.astype(i32),
        cat3.reshape(_B, 288),
    ], axis=1)
    seg_id = jnp.repeat(jnp.arange(5, dtype=i32),
                        jnp.asarray(_SEGL), total_repeat_length=_TPB)
    cnt1 = jnp.zeros((_B, 5), f32).at[
        jnp.arange(_B)[:, None], seg_id[None, :]].add(
        (tokp == 1).astype(f32))
    init = (-cnt1.reshape(-1)[:, None]) * emb_table[1][None, :]

    mesh = plsc.VectorSubcoreMesh(core_axis_name="c", subcore_axis_name="s")
    segsum = pl.kernel(
        _seg_body,
        mesh=mesh,
        compiler_params=pltpu.CompilerParams(use_tc_tiling_on_sc=False),
        out_type=jax.ShapeDtypeStruct((_B * 5, _D), f32),
        scratch_types=[
            pltpu.VMEM((_TPB,), i32),
            pltpu.VMEM((96, _D), f32),
            pltpu.VMEM((5, _D), f32),
            pltpu.SemaphoreType.DMA,
        ],
    )(emb_table, tokp.reshape(-1), init)

    cmask = category != 1
    any1 = jnp.any(cmask, axis=-1)
    n1 = any1.sum(-1).astype(f32)
    n2 = jnp.any(any1, axis=-1).sum(-1).astype(f32)
    w3 = 1.0 / (n1 + 1e-6) / (n2 + 1e-6)[:, None]
    wpad = jnp.pad(w3, ((0, 0), (0, 125)))

    ppad = jnp.pad(price.astype(f32), ((0, 0), (0, 128 - 11)))
    rpad = jnp.pad(rating.astype(f32), ((0, 0), (0, 128 - 4)))
    wppad = jnp.pad(W_price, ((0, 128 - 11), (0, 0)))
    wrpad = jnp.pad(W_rating, ((0, 128 - 4), (0, 0)))
    bb = 128
    small = pl.pallas_call(
        _small_body,
        grid=(_B // bb,),
        in_specs=[
            pl.BlockSpec((bb, 128), lambda i: (i, 0)),
            pl.BlockSpec((bb, 128), lambda i: (i, 0)),
            pl.BlockSpec((128, _D), lambda i: (0, 0)),
            pl.BlockSpec((128, _D), lambda i: (0, 0)),
            pl.BlockSpec((bb, 5, _D), lambda i: (i, 0, 0)),
            pl.BlockSpec((bb, 128), lambda i: (i, 0)),
            pl.BlockSpec((8, _D), lambda i: (0, 0)),
            pl.BlockSpec((1, _D), lambda i: (0, 0)),
            pl.BlockSpec((_D, _D), lambda i: (0, 0)),
            pl.BlockSpec((_D, _D), lambda i: (0, 0)),
        ],
        out_specs=pl.BlockSpec((bb, 5, _D), lambda i: (i, 0, 0)),
        out_shape=jax.ShapeDtypeStruct((_B, 5, _D), f32),
    )(ppad, rpad, wppad, wrpad, segsum.reshape(_B, 5, _D), wpad,
      nprj, bfc2d, wfc2, W_lin)


    out_flat = pl.kernel(
        _asm_body,
        mesh=mesh,
        compiler_params=pltpu.CompilerParams(use_tc_tiling_on_sc=False),
        out_type=jax.ShapeDtypeStruct((_B * 133, _D), f32),
        scratch_types=[
            pltpu.VMEM((64,), i32),
            pltpu.VMEM((64, _D), f32),
            pltpu.VMEM((5, _D), f32),
            pltpu.SemaphoreType.DMA,
        ],
    )(gtab, small.reshape(_B * 5, _D), description.reshape(-1).astype(i32))

    all_embeddings = out_flat.reshape(_B, 133, _D)
    all_masks = jnp.concatenate([
        price.sum(axis=1, keepdims=True) != 0.0,
        jnp.ones((_B, 1), bool),
        brand[:, :1] != 1,
        name[:, :1] != 1,
        jnp.ones((_B, 1), bool),
        description != 1,
    ], axis=1)
    return all_embeddings, all_masks


# speedup vs baseline: 3.0414x; 1.1263x over previous
import functools

import jax
import jax.numpy as jnp
from jax import lax
from jax.experimental import pallas as pl
from jax.experimental.pallas import tpu as pltpu
import jax.experimental.pallas.tpu_sc as plsc

_D = 1024
_V = 50265
_B = 1024

_NC = 2
_NS = 16
_NW = _NC * _NS
_BPW = _B // _NW
_SEGL = (16, 32, 96, 96, 96)
_SEGO = (0, 16, 48, 144, 240)
_TPB = 336

_GROWS = 512
_VPAD = 51200
_GBLK = _VPAD // _GROWS
_GEXT = _VPAD + _B * 5
_RPW = _B * 133 // _NW
_SPAN = 56


def _np_body(fnp_ref, wfc1_ref, out_ref):
    out_ref[...] = jnp.dot(fnp_ref[...], wfc1_ref[...],
                           preferred_element_type=jnp.float32)


def _gtab_body(x_ref, wfc2_ref, wlin_ref, np_ref, bfc_ref, out_ref):
    h = jnp.dot(x_ref[...], wfc2_ref[...], preferred_element_type=jnp.float32)
    h = jnp.maximum(h + np_ref[5:6, :] + bfc_ref[...], 0.0)
    out_ref[...] = jnp.dot(h, wlin_ref[...], preferred_element_type=jnp.float32)


def _small_body(g_ref, pp_ref, rp_ref, wp_ref, wr_ref, s_ref, w_ref, np_ref,
                bfc_ref, wfc2_ref, wlin_ref, out_ref):
    del g_ref
    pv = jnp.dot(pp_ref[...], wp_ref[...], preferred_element_type=jnp.float32)
    rv = jnp.dot(rp_ref[...], wr_ref[...], preferred_element_type=jnp.float32)
    bs = s_ref[:, 0, :]
    ns = s_ref[:, 1, :]
    cv = (w_ref[:, 0:1] * s_ref[:, 2, :] + w_ref[:, 1:2] * s_ref[:, 3, :]
          + w_ref[:, 2:3] * s_ref[:, 4, :])
    v = jnp.stack((pv, rv, bs, ns, cv), axis=1).reshape(-1, _D)
    h = jnp.dot(v, wfc2_ref[...], preferred_element_type=jnp.float32)
    h = jnp.maximum(h + jnp.tile(np_ref[0:5, :], (pv.shape[0], 1))
                    + bfc_ref[...], 0.0)
    out_ref[...] = jnp.dot(h, wlin_ref[...], preferred_element_type=jnp.float32)


def _seg_body(emb_hbm, tok_hbm, init_hbm, out_hbm, idx_v, buf_v, acc_v, sem):
    wid = lax.axis_index("s") * _NC + lax.axis_index("c")

    def b_body(j, carry):
        b = wid * _BPW + j
        pltpu.sync_copy(tok_hbm.at[pl.ds(b * _TPB, _TPB)], idx_v)
        pltpu.sync_copy(init_hbm.at[pl.ds(b * 5, 5)], acc_v)
        for s in range(5):
            n = _SEGL[s]
            pltpu.async_copy(emb_hbm.at[idx_v.at[pl.ds(_SEGO[s], n)]],
                             buf_v.at[pl.ds(0, n)], sem).wait()

            def cb_body(cb, c2):
                base = pl.multiple_of(cb * 256, 256)

                def r_body(r, accs):
                    return tuple(accs[k] + buf_v[r, pl.ds(base + k * 16, 16)]
                                 for k in range(16))

                acc0 = tuple(acc_v[s, pl.ds(base + k * 16, 16)]
                             for k in range(16))
                accs = lax.fori_loop(0, n, r_body, acc0)
                for k in range(16):
                    acc_v[s, pl.ds(base + k * 16, 16)] = accs[k]
                return c2

            lax.fori_loop(0, _D // 256, cb_body, 0)
        pltpu.sync_copy(acc_v, out_hbm.at[pl.ds(b * 5, 5)])
        return carry

    lax.fori_loop(0, _BPW, b_body, 0)


def _asm_body(g_hbm, ridx_hbm, out_hbm, idx_v, rows_v, sem):
    wid = lax.axis_index("s") * _NC + lax.axis_index("c")

    def sp_body(i, carry):
        off = wid * _RPW + i * _SPAN
        pltpu.sync_copy(ridx_hbm.at[pl.ds(off, _SPAN)], idx_v)
        pltpu.async_copy(g_hbm.at[idx_v], rows_v, sem).wait()
        pltpu.sync_copy(rows_v, out_hbm.at[pl.ds(off, _SPAN)])
        return carry

    lax.fori_loop(0, _RPW // _SPAN, sp_body, 0)


def kernel(field, price, rating, brand, name, category, description,
           emb_table, W_price, W_rating, W_fc, b_fc, W_lin):
    f32 = jnp.float32
    i32 = jnp.int32
    wfc1 = W_fc[:_D]
    wfc2 = W_fc[_D:]
    bfc2d = b_fc.reshape(1, _D)

    fnp = jnp.take(emb_table, field[:, 0], axis=0)
    fnp = jnp.concatenate([fnp, jnp.zeros((2, _D), f32)], axis=0)
    nprj = pl.pallas_call(
        _np_body,
        out_shape=jax.ShapeDtypeStruct((8, _D), f32),
    )(fnp, wfc1)

    gtab = pl.pallas_call(
        _gtab_body,
        grid=(_GBLK,),
        in_specs=[
            pl.BlockSpec((_GROWS, _D), lambda i: (jnp.minimum(i, _GBLK - 2), 0)),
            pl.BlockSpec((_D, _D), lambda i: (0, 0)),
            pl.BlockSpec((_D, _D), lambda i: (0, 0)),
            pl.BlockSpec((8, _D), lambda i: (0, 0)),
            pl.BlockSpec((1, _D), lambda i: (0, 0)),
        ],
        out_specs=pl.BlockSpec((_GROWS, _D), lambda i: (i, 0)),
        out_shape=jax.ShapeDtypeStruct((_GEXT, _D), f32),
    )(emb_table, wfc2, W_lin, nprj, bfc2d)

    cat3 = category.reshape(_B, 3, 96).astype(i32)
    tokp = jnp.concatenate([
        jnp.pad(brand.astype(i32), ((0, 0), (0, 4)), constant_values=1),
        name.astype(i32),
        cat3.reshape(_B, 288),
    ], axis=1)
    tok1 = (tokp == 1).astype(f32)
    cnt1 = jnp.stack([
        tok1[:, 0:16].sum(1), tok1[:, 16:48].sum(1), tok1[:, 48:144].sum(1),
        tok1[:, 144:240].sum(1), tok1[:, 240:336].sum(1)], axis=1)
    init = (-cnt1.reshape(-1)[:, None]) * emb_table[1][None, :]

    mesh = plsc.VectorSubcoreMesh(core_axis_name="c", subcore_axis_name="s")
    segsum = pl.kernel(
        _seg_body,
        mesh=mesh,
        compiler_params=pltpu.CompilerParams(use_tc_tiling_on_sc=False),
        out_type=jax.ShapeDtypeStruct((_B * 5, _D), f32),
        scratch_types=[
            pltpu.VMEM((_TPB,), i32),
            pltpu.VMEM((96, _D), f32),
            pltpu.VMEM((5, _D), f32),
            pltpu.SemaphoreType.DMA,
        ],
    )(emb_table, tokp.reshape(-1), init)

    cmask = category != 1
    any1 = jnp.any(cmask, axis=-1)
    n1 = any1.sum(-1).astype(f32)
    n2 = jnp.any(any1, axis=-1).sum(-1).astype(f32)
    w3 = 1.0 / (n1 + 1e-6) / (n2 + 1e-6)[:, None]
    wpad = jnp.pad(w3, ((0, 0), (0, 125)))

    ppad = jnp.pad(price.astype(f32), ((0, 0), (0, 128 - 11)))
    rpad = jnp.pad(rating.astype(f32), ((0, 0), (0, 128 - 4)))
    wppad = jnp.pad(W_price, ((0, 128 - 11), (0, 0)))
    wrpad = jnp.pad(W_rating, ((0, 128 - 4), (0, 0)))
    bb = 128
    gext = pl.pallas_call(
        _small_body,
        grid=(_B // bb,),
        in_specs=[
            pl.BlockSpec((8, _D), lambda i: (0, 0)),
            pl.BlockSpec((bb, 128), lambda i: (i, 0)),
            pl.BlockSpec((bb, 128), lambda i: (i, 0)),
            pl.BlockSpec((128, _D), lambda i: (0, 0)),
            pl.BlockSpec((128, _D), lambda i: (0, 0)),
            pl.BlockSpec((bb, 5, _D), lambda i: (i, 0, 0)),
            pl.BlockSpec((bb, 128), lambda i: (i, 0)),
            pl.BlockSpec((8, _D), lambda i: (0, 0)),
            pl.BlockSpec((1, _D), lambda i: (0, 0)),
            pl.BlockSpec((_D, _D), lambda i: (0, 0)),
            pl.BlockSpec((_D, _D), lambda i: (0, 0)),
        ],
        out_specs=pl.BlockSpec((bb * 5, _D), lambda i: (_VPAD // (bb * 5) + i, 0)),
        out_shape=jax.ShapeDtypeStruct((_GEXT, _D), f32),
        input_output_aliases={0: 0},
    )(gtab, ppad, rpad, wppad, wrpad, segsum.reshape(_B, 5, _D), wpad,
      nprj, bfc2d, wfc2, W_lin)

    small_idx = (_VPAD + 5 * jnp.arange(_B, dtype=i32))[:, None] \
        + jnp.arange(5, dtype=i32)[None, :]
    ridx = jnp.concatenate([small_idx, description.astype(i32)],
                           axis=1).reshape(-1)
    out_flat = pl.kernel(
        _asm_body,
        mesh=mesh,
        out_type=jax.ShapeDtypeStruct((_B * 133, _D), f32),
        scratch_types=[
            pltpu.VMEM((_SPAN,), i32),
            pltpu.VMEM((_SPAN, _D), f32),
            pltpu.SemaphoreType.DMA,
        ],
    )(gext, ridx)

    all_embeddings = out_flat.reshape(_B, 133, _D)
    all_masks = jnp.concatenate([
        price.sum(axis=1, keepdims=True) != 0.0,
        jnp.ones((_B, 1), bool),
        brand[:, :1] != 1,
        name[:, :1] != 1,
        jnp.ones((_B, 1), bool),
        description != 1,
    ], axis=1)
    return all_embeddings, all_masks


# speedup vs baseline: 3.3114x; 1.0888x over previous
import functools

import jax
import jax.numpy as jnp
from jax import lax
from jax.experimental import pallas as pl
from jax.experimental.pallas import tpu as pltpu
import jax.experimental.pallas.tpu_sc as plsc

_D = 1024
_V = 50265
_B = 1024

_NC = 2
_NS = 16
_NW = _NC * _NS
_BPW = _B // _NW
_SEGL = (16, 32, 96, 96, 96)
_SEGO = (0, 16, 48, 144, 240)
_TPB = 336

_GROWS = 512
_VPAD = 51200
_GBLK = _VPAD // _GROWS
_GEXT = _VPAD + _B * 5
_RPW = _B * 133 // _NW
_SPAN = 56


def _np_body(fnp_ref, wfc1_ref, out_ref):
    out_ref[...] = jnp.dot(fnp_ref[...], wfc1_ref[...],
                           preferred_element_type=jnp.float32)


def _gtab_body(x_ref, wfc2_ref, wlin_ref, np_ref, bfc_ref, out_ref):
    h = jnp.dot(x_ref[...].astype(jnp.bfloat16), wfc2_ref[...],
                preferred_element_type=jnp.float32)
    h = jnp.maximum(h + np_ref[5:6, :] + bfc_ref[...], 0.0)
    out_ref[...] = jnp.dot(h.astype(jnp.bfloat16), wlin_ref[...],
                           preferred_element_type=jnp.float32)


def _small_body(g_ref, pp_ref, rp_ref, wp_ref, wr_ref, s_ref, w_ref, np_ref,
                bfc_ref, wfc2_ref, wlin_ref, out_ref):
    del g_ref
    pv = jnp.dot(pp_ref[...], wp_ref[...], preferred_element_type=jnp.float32)
    rv = jnp.dot(rp_ref[...], wr_ref[...], preferred_element_type=jnp.float32)
    bs = s_ref[:, 0, :]
    ns = s_ref[:, 1, :]
    cv = (w_ref[:, 0:1] * s_ref[:, 2, :] + w_ref[:, 1:2] * s_ref[:, 3, :]
          + w_ref[:, 2:3] * s_ref[:, 4, :])
    v = jnp.stack((pv, rv, bs, ns, cv), axis=1).reshape(-1, _D)
    h = jnp.dot(v, wfc2_ref[...], preferred_element_type=jnp.float32)
    h = jnp.maximum(h + jnp.tile(np_ref[0:5, :], (pv.shape[0], 1))
                    + bfc_ref[...], 0.0)
    out_ref[...] = jnp.dot(h, wlin_ref[...], preferred_element_type=jnp.float32)


_CHUNKS = ((0, 0, 16), (1, 16, 32), (2, 48, 48), (2, 96, 48),
           (3, 144, 48), (3, 192, 48), (4, 240, 48), (4, 288, 48))


def _seg_body(emb_hbm, tok_hbm, init_hbm, out_hbm,
              idx_v, bufa_v, bufb_v, acc_v, sem_a, sem_b):
    wid = lax.axis_index("s") * _NC + lax.axis_index("c")
    bufs = (bufa_v, bufb_v)
    sems = (sem_a, sem_b)

    def b_body(j, carry):
        b = wid * _BPW + j
        pltpu.sync_copy(tok_hbm.at[pl.ds(b * _TPB, _TPB)], idx_v)
        pltpu.sync_copy(init_hbm.at[pl.ds(b * 5, 5)], acc_v)
        cps = {}
        s0, o0, n0 = _CHUNKS[0]
        cps[0] = pltpu.async_copy(emb_hbm.at[idx_v.at[pl.ds(o0, n0)]],
                                  bufa_v.at[pl.ds(0, n0)], sem_a)
        for i, (s, o, n) in enumerate(_CHUNKS):
            if i + 1 < len(_CHUNKS):
                s2, o2, n2 = _CHUNKS[i + 1]
                cps[i + 1] = pltpu.async_copy(
                    emb_hbm.at[idx_v.at[pl.ds(o2, n2)]],
                    bufs[(i + 1) % 2].at[pl.ds(0, n2)], sems[(i + 1) % 2])
            cps[i].wait()
            buf = bufs[i % 2]

            def cb_body(cb, c2, s=s, n=n, buf=buf):
                base = pl.multiple_of(cb * 256, 256)

                def r_body(r, accs):
                    return tuple(accs[k] + buf[r, pl.ds(base + k * 16, 16)]
                                 for k in range(16))

                acc0 = tuple(acc_v[s, pl.ds(base + k * 16, 16)]
                             for k in range(16))
                accs = lax.fori_loop(0, n, r_body, acc0)
                for k in range(16):
                    acc_v[s, pl.ds(base + k * 16, 16)] = accs[k]
                return c2

            lax.fori_loop(0, _D // 256, cb_body, 0)
        pltpu.sync_copy(acc_v, out_hbm.at[pl.ds(b * 5, 5)])
        return carry

    lax.fori_loop(0, _BPW, b_body, 0)


def _asm_body(g_hbm, ridx_hbm, out_hbm, idx_v, rows_v, sem):
    wid = lax.axis_index("s") * _NC + lax.axis_index("c")

    def sp_body(i, carry):
        off = wid * _RPW + i * _SPAN
        pltpu.sync_copy(ridx_hbm.at[pl.ds(off, _SPAN)], idx_v)
        pltpu.async_copy(g_hbm.at[idx_v], rows_v, sem).wait()
        pltpu.sync_copy(rows_v, out_hbm.at[pl.ds(off, _SPAN)])
        return carry

    lax.fori_loop(0, _RPW // _SPAN, sp_body, 0)


def kernel(field, price, rating, brand, name, category, description,
           emb_table, W_price, W_rating, W_fc, b_fc, W_lin):
    f32 = jnp.float32
    i32 = jnp.int32
    wfc1 = W_fc[:_D]
    wfc2 = W_fc[_D:]
    bfc2d = b_fc.reshape(1, _D)

    fnp = jnp.take(emb_table, field[:, 0], axis=0)
    fnp = jnp.concatenate([fnp, jnp.zeros((2, _D), f32)], axis=0)
    nprj = pl.pallas_call(
        _np_body,
        out_shape=jax.ShapeDtypeStruct((8, _D), f32),
    )(fnp, wfc1)

    gtab = pl.pallas_call(
        _gtab_body,
        grid=(_GBLK,),
        in_specs=[
            pl.BlockSpec((_GROWS, _D), lambda i: (jnp.minimum(i, _GBLK - 2), 0)),
            pl.BlockSpec((_D, _D), lambda i: (0, 0)),
            pl.BlockSpec((_D, _D), lambda i: (0, 0)),
            pl.BlockSpec((8, _D), lambda i: (0, 0)),
            pl.BlockSpec((1, _D), lambda i: (0, 0)),
        ],
        out_specs=pl.BlockSpec((_GROWS, _D), lambda i: (i, 0)),
        out_shape=jax.ShapeDtypeStruct((_GEXT, _D), f32),
    )(emb_table, wfc2.astype(jnp.bfloat16), W_lin.astype(jnp.bfloat16),
      nprj, bfc2d)

    cat3 = category.reshape(_B, 3, 96).astype(i32)
    tokp = jnp.concatenate([
        jnp.pad(brand.astype(i32), ((0, 0), (0, 4)), constant_values=1),
        name.astype(i32),
        cat3.reshape(_B, 288),
    ], axis=1)
    tok1 = (tokp == 1).astype(f32)
    cnt1 = jnp.stack([
        tok1[:, 0:16].sum(1), tok1[:, 16:48].sum(1), tok1[:, 48:144].sum(1),
        tok1[:, 144:240].sum(1), tok1[:, 240:336].sum(1)], axis=1)
    init = (-cnt1.reshape(-1)[:, None]) * emb_table[1][None, :]

    mesh = plsc.VectorSubcoreMesh(core_axis_name="c", subcore_axis_name="s")
    segsum = pl.kernel(
        _seg_body,
        mesh=mesh,
        compiler_params=pltpu.CompilerParams(use_tc_tiling_on_sc=False),
        out_type=jax.ShapeDtypeStruct((_B * 5, _D), f32),
        scratch_types=[
            pltpu.VMEM((_TPB,), i32),
            pltpu.VMEM((48, _D), f32),
            pltpu.VMEM((48, _D), f32),
            pltpu.VMEM((5, _D), f32),
            pltpu.SemaphoreType.DMA,
            pltpu.SemaphoreType.DMA,
        ],
    )(emb_table, tokp.reshape(-1), init)

    cmask = category != 1
    any1 = jnp.any(cmask, axis=-1)
    n1 = any1.sum(-1).astype(f32)
    n2 = jnp.any(any1, axis=-1).sum(-1).astype(f32)
    w3 = 1.0 / (n1 + 1e-6) / (n2 + 1e-6)[:, None]
    wpad = jnp.pad(w3, ((0, 0), (0, 125)))

    ppad = jnp.pad(price.astype(f32), ((0, 0), (0, 128 - 11)))
    rpad = jnp.pad(rating.astype(f32), ((0, 0), (0, 128 - 4)))
    wppad = jnp.pad(W_price, ((0, 128 - 11), (0, 0)))
    wrpad = jnp.pad(W_rating, ((0, 128 - 4), (0, 0)))
    bb = 128
    gext = pl.pallas_call(
        _small_body,
        grid=(_B // bb,),
        in_specs=[
            pl.BlockSpec((8, _D), lambda i: (0, 0)),
            pl.BlockSpec((bb, 128), lambda i: (i, 0)),
            pl.BlockSpec((bb, 128), lambda i: (i, 0)),
            pl.BlockSpec((128, _D), lambda i: (0, 0)),
            pl.BlockSpec((128, _D), lambda i: (0, 0)),
            pl.BlockSpec((bb, 5, _D), lambda i: (i, 0, 0)),
            pl.BlockSpec((bb, 128), lambda i: (i, 0)),
            pl.BlockSpec((8, _D), lambda i: (0, 0)),
            pl.BlockSpec((1, _D), lambda i: (0, 0)),
            pl.BlockSpec((_D, _D), lambda i: (0, 0)),
            pl.BlockSpec((_D, _D), lambda i: (0, 0)),
        ],
        out_specs=pl.BlockSpec((bb * 5, _D), lambda i: (_VPAD // (bb * 5) + i, 0)),
        out_shape=jax.ShapeDtypeStruct((_GEXT, _D), f32),
        input_output_aliases={0: 0},
    )(gtab, ppad, rpad, wppad, wrpad, segsum.reshape(_B, 5, _D), wpad,
      nprj, bfc2d, wfc2, W_lin)

    small_idx = (_VPAD + 5 * jnp.arange(_B, dtype=i32))[:, None] \
        + jnp.arange(5, dtype=i32)[None, :]
    ridx = jnp.concatenate([small_idx, description.astype(i32)],
                           axis=1).reshape(-1)
    out_flat = pl.kernel(
        _asm_body,
        mesh=mesh,
        out_type=jax.ShapeDtypeStruct((_B * 133, _D), f32),
        scratch_types=[
            pltpu.VMEM((_SPAN,), i32),
            pltpu.VMEM((_SPAN, _D), f32),
            pltpu.SemaphoreType.DMA,
        ],
    )(gext, ridx)
    all_embeddings = out_flat.reshape(_B, 133, _D)
    all_masks = jnp.concatenate([
        price.sum(axis=1, keepdims=True) != 0.0,
        jnp.ones((_B, 1), bool),
        brand[:, :1] != 1,
        name[:, :1] != 1,
        jnp.ones((_B, 1), bool),
        description != 1,
    ], axis=1)
    return all_embeddings, all_masks


# speedup vs baseline: 3.4111x; 1.0301x over previous
import functools

import jax
import jax.numpy as jnp
from jax import lax
from jax.experimental import pallas as pl
from jax.experimental.pallas import tpu as pltpu
import jax.experimental.pallas.tpu_sc as plsc

_D = 1024
_V = 50265
_B = 1024

_NC = 2
_NS = 16
_NW = _NC * _NS
_BPW = _B // _NW
_SEGL = (16, 32, 96, 96, 96)
_SEGO = (0, 16, 48, 144, 240)
_TPB = 336

_GROWS = 512
_VPAD = 51200
_GBLK = _VPAD // _GROWS
_GEXT = _VPAD + _B * 5
_RPW = _B * 133 // _NW
_SPAN = 56


def _np_body(fnp_ref, wfc1_ref, out_ref):
    out_ref[...] = jnp.dot(fnp_ref[...], wfc1_ref[...],
                           preferred_element_type=jnp.float32)


def _gtab_body(x_ref, wfc2_ref, wlin_ref, np_ref, bfc_ref, out_ref):
    h = jnp.dot(x_ref[...].astype(jnp.bfloat16), wfc2_ref[...],
                preferred_element_type=jnp.float32)
    h = jnp.maximum(h + np_ref[5:6, :] + bfc_ref[...], 0.0)
    out_ref[...] = jnp.dot(h.astype(jnp.bfloat16), wlin_ref[...],
                           preferred_element_type=jnp.float32)


def _small_body(g_ref, pp_ref, rp_ref, wp_ref, wr_ref, s_ref, w_ref, np_ref,
                bfc_ref, wfc2_ref, wlin_ref, out_ref):
    del g_ref
    pv = jnp.dot(pp_ref[...], wp_ref[...], preferred_element_type=jnp.float32)
    rv = jnp.dot(rp_ref[...], wr_ref[...], preferred_element_type=jnp.float32)
    bs = s_ref[:, 0, :]
    ns = s_ref[:, 1, :]
    cv = (w_ref[:, 0:1] * s_ref[:, 2, :] + w_ref[:, 1:2] * s_ref[:, 3, :]
          + w_ref[:, 2:3] * s_ref[:, 4, :])
    v = jnp.stack((pv, rv, bs, ns, cv), axis=1).reshape(-1, _D)
    h = jnp.dot(v, wfc2_ref[...], preferred_element_type=jnp.float32)
    h = jnp.maximum(h + jnp.tile(np_ref[0:5, :], (pv.shape[0], 1))
                    + bfc_ref[...], 0.0)
    out_ref[...] = jnp.dot(h, wlin_ref[...], preferred_element_type=jnp.float32)


_CHUNKS = ((0, 0, 16), (1, 16, 32), (2, 48, 48), (2, 96, 48),
           (3, 144, 48), (3, 192, 48), (4, 240, 48), (4, 288, 48))


def _seg_body(emb_hbm, tok_hbm, init_hbm, out_hbm,
              idx_v, bufa_v, bufb_v, acc_v, sem_a, sem_b):
    wid = lax.axis_index("s") * _NC + lax.axis_index("c")
    bufs = (bufa_v, bufb_v)
    sems = (sem_a, sem_b)

    def b_body(j, carry):
        b = wid * _BPW + j
        pltpu.sync_copy(tok_hbm.at[pl.ds(b * _TPB, _TPB)], idx_v)
        pltpu.sync_copy(init_hbm.at[pl.ds(b * 5, 5)], acc_v)
        cps = {}
        s0, o0, n0 = _CHUNKS[0]
        cps[0] = pltpu.async_copy(emb_hbm.at[idx_v.at[pl.ds(o0, n0)]],
                                  bufa_v.at[pl.ds(0, n0)], sem_a)
        for i, (s, o, n) in enumerate(_CHUNKS):
            if i + 1 < len(_CHUNKS):
                s2, o2, n2 = _CHUNKS[i + 1]
                cps[i + 1] = pltpu.async_copy(
                    emb_hbm.at[idx_v.at[pl.ds(o2, n2)]],
                    bufs[(i + 1) % 2].at[pl.ds(0, n2)], sems[(i + 1) % 2])
            cps[i].wait()
            buf = bufs[i % 2]

            def cb_body(cb, c2, s=s, n=n, buf=buf):
                base = pl.multiple_of(cb * 256, 256)

                def r_body(r, accs):
                    return tuple(accs[k] + buf[r, pl.ds(base + k * 16, 16)]
                                 for k in range(16))

                acc0 = tuple(acc_v[s, pl.ds(base + k * 16, 16)]
                             for k in range(16))
                accs = lax.fori_loop(0, n, r_body, acc0)
                for k in range(16):
                    acc_v[s, pl.ds(base + k * 16, 16)] = accs[k]
                return c2

            lax.fori_loop(0, _D // 256, cb_body, 0)
        pltpu.sync_copy(acc_v, out_hbm.at[pl.ds(b * 5, 5)])
        return carry

    lax.fori_loop(0, _BPW, b_body, 0)


def _asm_body(g_hbm, ridx_hbm, out_hbm,
              idxa_v, idxb_v, rowsa_v, rowsb_v, sem_a, sem_b):
    wid = lax.axis_index("s") * _NC + lax.axis_index("c")
    base = wid * _RPW
    idxs = (idxa_v, idxb_v)
    rows = (rowsa_v, rowsb_v)
    sems = (sem_a, sem_b)
    nsp = _RPW // _SPAN

    pltpu.sync_copy(ridx_hbm.at[pl.ds(base, _SPAN)], idxa_v)
    pltpu.async_copy(g_hbm.at[idxa_v], rowsa_v, sem_a)

    def pair_body(p, carry):
        for k in range(2):
            i = p * 2 + k
            nk = (k + 1) % 2

            @pl.when(i + 1 < nsp)
            def _(i=i, nk=nk):
                pltpu.sync_copy(
                    ridx_hbm.at[pl.ds(base + (i + 1) * _SPAN, _SPAN)],
                    idxs[nk])
                pltpu.async_copy(g_hbm.at[idxs[nk]], rows[nk], sems[nk])

            pltpu.make_async_copy(g_hbm.at[idxs[k]], rows[k], sems[k]).wait()
            pltpu.sync_copy(rows[k], out_hbm.at[pl.ds(base + i * _SPAN,
                                                      _SPAN)])
        return carry

    lax.fori_loop(0, nsp // 2, pair_body, 0)


def kernel(field, price, rating, brand, name, category, description,
           emb_table, W_price, W_rating, W_fc, b_fc, W_lin):
    f32 = jnp.float32
    i32 = jnp.int32
    wfc1 = W_fc[:_D]
    wfc2 = W_fc[_D:]
    bfc2d = b_fc.reshape(1, _D)

    fnp = jnp.take(emb_table, field[:, 0], axis=0)
    fnp = jnp.concatenate([fnp, jnp.zeros((2, _D), f32)], axis=0)
    nprj = pl.pallas_call(
        _np_body,
        out_shape=jax.ShapeDtypeStruct((8, _D), f32),
    )(fnp, wfc1)

    gtab = pl.pallas_call(
        _gtab_body,
        grid=(_GBLK,),
        in_specs=[
            pl.BlockSpec((_GROWS, _D), lambda i: (jnp.minimum(i, _GBLK - 2), 0)),
            pl.BlockSpec((_D, _D), lambda i: (0, 0)),
            pl.BlockSpec((_D, _D), lambda i: (0, 0)),
            pl.BlockSpec((8, _D), lambda i: (0, 0)),
            pl.BlockSpec((1, _D), lambda i: (0, 0)),
        ],
        out_specs=pl.BlockSpec((_GROWS, _D), lambda i: (i, 0)),
        out_shape=jax.ShapeDtypeStruct((_GEXT, _D), f32),
    )(emb_table, wfc2.astype(jnp.bfloat16), W_lin.astype(jnp.bfloat16),
      nprj, bfc2d)

    cat3 = category.reshape(_B, 3, 96).astype(i32)
    tokp = jnp.concatenate([
        jnp.pad(brand.astype(i32), ((0, 0), (0, 4)), constant_values=1),
        name.astype(i32),
        cat3.reshape(_B, 288),
    ], axis=1)
    tok1 = (tokp == 1).astype(f32)
    cnt1 = jnp.stack([
        tok1[:, 0:16].sum(1), tok1[:, 16:48].sum(1), tok1[:, 48:144].sum(1),
        tok1[:, 144:240].sum(1), tok1[:, 240:336].sum(1)], axis=1)
    init = (-cnt1.reshape(-1)[:, None]) * emb_table[1][None, :]

    mesh = plsc.VectorSubcoreMesh(core_axis_name="c", subcore_axis_name="s")
    segsum = pl.kernel(
        _seg_body,
        mesh=mesh,
        compiler_params=pltpu.CompilerParams(use_tc_tiling_on_sc=False),
        out_type=jax.ShapeDtypeStruct((_B * 5, _D), f32),
        scratch_types=[
            pltpu.VMEM((_TPB,), i32),
            pltpu.VMEM((48, _D), f32),
            pltpu.VMEM((48, _D), f32),
            pltpu.VMEM((5, _D), f32),
            pltpu.SemaphoreType.DMA,
            pltpu.SemaphoreType.DMA,
        ],
    )(emb_table, tokp.reshape(-1), init)

    cmask = category != 1
    any1 = jnp.any(cmask, axis=-1)
    n1 = any1.sum(-1).astype(f32)
    n2 = jnp.any(any1, axis=-1).sum(-1).astype(f32)
    w3 = 1.0 / (n1 + 1e-6) / (n2 + 1e-6)[:, None]
    wpad = jnp.pad(w3, ((0, 0), (0, 125)))

    ppad = jnp.pad(price.astype(f32), ((0, 0), (0, 128 - 11)))
    rpad = jnp.pad(rating.astype(f32), ((0, 0), (0, 128 - 4)))
    wppad = jnp.pad(W_price, ((0, 128 - 11), (0, 0)))
    wrpad = jnp.pad(W_rating, ((0, 128 - 4), (0, 0)))
    bb = 128
    gext = pl.pallas_call(
        _small_body,
        grid=(_B // bb,),
        in_specs=[
            pl.BlockSpec((8, _D), lambda i: (0, 0)),
            pl.BlockSpec((bb, 128), lambda i: (i, 0)),
            pl.BlockSpec((bb, 128), lambda i: (i, 0)),
            pl.BlockSpec((128, _D), lambda i: (0, 0)),
            pl.BlockSpec((128, _D), lambda i: (0, 0)),
            pl.BlockSpec((bb, 5, _D), lambda i: (i, 0, 0)),
            pl.BlockSpec((bb, 128), lambda i: (i, 0)),
            pl.BlockSpec((8, _D), lambda i: (0, 0)),
            pl.BlockSpec((1, _D), lambda i: (0, 0)),
            pl.BlockSpec((_D, _D), lambda i: (0, 0)),
            pl.BlockSpec((_D, _D), lambda i: (0, 0)),
        ],
        out_specs=pl.BlockSpec((bb * 5, _D), lambda i: (_VPAD // (bb * 5) + i, 0)),
        out_shape=jax.ShapeDtypeStruct((_GEXT, _D), f32),
        input_output_aliases={0: 0},
    )(gtab, ppad, rpad, wppad, wrpad, segsum.reshape(_B, 5, _D), wpad,
      nprj, bfc2d, wfc2, W_lin)

    small_idx = (_VPAD + 5 * jnp.arange(_B, dtype=i32))[:, None] \
        + jnp.arange(5, dtype=i32)[None, :]
    ridx = jnp.concatenate([small_idx, description.astype(i32)],
                           axis=1).reshape(-1)
    out_flat = pl.kernel(
        _asm_body,
        mesh=mesh,
        out_type=jax.ShapeDtypeStruct((_B * 133, _D), f32),
        scratch_types=[
            pltpu.VMEM((_SPAN,), i32),
            pltpu.VMEM((_SPAN,), i32),
            pltpu.VMEM((_SPAN, _D), f32),
            pltpu.VMEM((_SPAN, _D), f32),
            pltpu.SemaphoreType.DMA,
            pltpu.SemaphoreType.DMA,
        ],
    )(gext, ridx)
    all_embeddings = out_flat.reshape(_B, 133, _D)
    all_masks = jnp.concatenate([
        price.sum(axis=1, keepdims=True) != 0.0,
        jnp.ones((_B, 1), bool),
        brand[:, :1] != 1,
        name[:, :1] != 1,
        jnp.ones((_B, 1), bool),
        description != 1,
    ], axis=1)
    return all_embeddings, all_masks


# speedup vs baseline: 3.4128x; 1.0005x over previous
import functools

import jax
import jax.numpy as jnp
from jax import lax
from jax.experimental import pallas as pl
from jax.experimental.pallas import tpu as pltpu
import jax.experimental.pallas.tpu_sc as plsc

_D = 1024
_V = 50265
_B = 1024

_NC = 2
_NS = 16
_NW = _NC * _NS
_BPW = _B // _NW
_SEGL = (16, 32, 96, 96, 96)
_SEGO = (0, 16, 48, 144, 240)
_TPB = 336

_GROWS = 512
_VPAD = 51200
_GBLK = _VPAD // _GROWS
_GEXT = _VPAD + _B * 5
_RPW = _B * 133 // _NW
_SPAN = 56


def _np_body(fnp_ref, wfc1_ref, out_ref):
    out_ref[...] = jnp.dot(fnp_ref[...], wfc1_ref[...],
                           preferred_element_type=jnp.float32)


def _gtab_body(x_ref, wfc2_ref, wlin_ref, np_ref, bfc_ref, out_ref):
    h = jnp.dot(x_ref[...].astype(jnp.bfloat16), wfc2_ref[...],
                preferred_element_type=jnp.float32)
    h = jnp.maximum(h + np_ref[5:6, :] + bfc_ref[...], 0.0)
    out_ref[...] = jnp.dot(h.astype(jnp.bfloat16), wlin_ref[...],
                           preferred_element_type=jnp.float32)


def _small_body(g_ref, pp_ref, rp_ref, wp_ref, wr_ref, s_ref, w_ref, np_ref,
                bfc_ref, wfc2_ref, wlin_ref, out_ref):
    del g_ref
    pv = jnp.dot(pp_ref[...], wp_ref[...], preferred_element_type=jnp.float32)
    rv = jnp.dot(rp_ref[...], wr_ref[...], preferred_element_type=jnp.float32)
    bs = s_ref[:, 0, :]
    ns = s_ref[:, 1, :]
    cv = (w_ref[:, 0:1] * s_ref[:, 2, :] + w_ref[:, 1:2] * s_ref[:, 3, :]
          + w_ref[:, 2:3] * s_ref[:, 4, :])
    v = jnp.stack((pv, rv, bs, ns, cv), axis=1).reshape(-1, _D)
    h = jnp.dot(v, wfc2_ref[...], preferred_element_type=jnp.float32)
    h = jnp.maximum(h + jnp.tile(np_ref[0:5, :], (pv.shape[0], 1))
                    + bfc_ref[...], 0.0)
    out_ref[...] = jnp.dot(h, wlin_ref[...], preferred_element_type=jnp.float32)


_CHUNKS = ((0, 0, 16), (1, 16, 32), (2, 48, 48), (2, 96, 48),
           (3, 144, 48), (3, 192, 48), (4, 240, 48), (4, 288, 48))


def _seg_body(emb_hbm, tok_hbm, init_hbm, out_hbm,
              idx_v, bufa_v, bufb_v, acc_v, sem_a, sem_b):
    wid = lax.axis_index("s") * _NC + lax.axis_index("c")
    bufs = (bufa_v, bufb_v)
    sems = (sem_a, sem_b)

    def b_body(j, carry):
        b = wid * _BPW + j
        pltpu.sync_copy(tok_hbm.at[pl.ds(b * _TPB, _TPB)], idx_v)
        pltpu.sync_copy(init_hbm.at[pl.ds(b * 5, 5)], acc_v)
        cps = {}
        s0, o0, n0 = _CHUNKS[0]
        cps[0] = pltpu.async_copy(emb_hbm.at[idx_v.at[pl.ds(o0, n0)]],
                                  bufa_v.at[pl.ds(0, n0)], sem_a)
        for i, (s, o, n) in enumerate(_CHUNKS):
            if i + 1 < len(_CHUNKS):
                s2, o2, n2 = _CHUNKS[i + 1]
                cps[i + 1] = pltpu.async_copy(
                    emb_hbm.at[idx_v.at[pl.ds(o2, n2)]],
                    bufs[(i + 1) % 2].at[pl.ds(0, n2)], sems[(i + 1) % 2])
            cps[i].wait()
            buf = bufs[i % 2]

            def cb_body(cb, c2, s=s, n=n, buf=buf):
                base = pl.multiple_of(cb * 256, 256)

                def r_body(r, accs):
                    t = tuple(accs[k] + buf[2 * r, pl.ds(base + k * 16, 16)]
                              for k in range(16))
                    return tuple(t[k] + buf[2 * r + 1,
                                            pl.ds(base + k * 16, 16)]
                                 for k in range(16))

                acc0 = tuple(acc_v[s, pl.ds(base + k * 16, 16)]
                             for k in range(16))
                accs = lax.fori_loop(0, n // 2, r_body, acc0)
                for k in range(16):
                    acc_v[s, pl.ds(base + k * 16, 16)] = accs[k]
                return c2

            lax.fori_loop(0, _D // 256, cb_body, 0)
        pltpu.sync_copy(acc_v, out_hbm.at[pl.ds(b * 5, 5)])
        return carry

    lax.fori_loop(0, _BPW, b_body, 0)


def _asm_body(g_hbm, ridx_hbm, out_hbm,
              idxa_v, idxb_v, rowsa_v, rowsb_v, sem_a, sem_b):
    wid = lax.axis_index("s") * _NC + lax.axis_index("c")
    base = wid * _RPW
    idxs = (idxa_v, idxb_v)
    rows = (rowsa_v, rowsb_v)
    sems = (sem_a, sem_b)
    nsp = _RPW // _SPAN

    pltpu.sync_copy(ridx_hbm.at[pl.ds(base, _SPAN)], idxa_v)
    pltpu.async_copy(g_hbm.at[idxa_v], rowsa_v, sem_a)

    def pair_body(p, carry):
        for k in range(2):
            i = p * 2 + k
            nk = (k + 1) % 2

            @pl.when(i + 1 < nsp)
            def _(i=i, nk=nk):
                pltpu.sync_copy(
                    ridx_hbm.at[pl.ds(base + (i + 1) * _SPAN, _SPAN)],
                    idxs[nk])
                pltpu.async_copy(g_hbm.at[idxs[nk]], rows[nk], sems[nk])

            pltpu.make_async_copy(g_hbm.at[idxs[k]], rows[k], sems[k]).wait()
            pltpu.sync_copy(rows[k], out_hbm.at[pl.ds(base + i * _SPAN,
                                                      _SPAN)])
        return carry

    lax.fori_loop(0, nsp // 2, pair_body, 0)


def kernel(field, price, rating, brand, name, category, description,
           emb_table, W_price, W_rating, W_fc, b_fc, W_lin):
    f32 = jnp.float32
    i32 = jnp.int32
    wfc1 = W_fc[:_D]
    wfc2 = W_fc[_D:]
    bfc2d = b_fc.reshape(1, _D)

    fnp = jnp.take(emb_table, field[:, 0], axis=0)
    fnp = jnp.concatenate([fnp, jnp.zeros((2, _D), f32)], axis=0)
    nprj = pl.pallas_call(
        _np_body,
        out_shape=jax.ShapeDtypeStruct((8, _D), f32),
    )(fnp, wfc1)

    gtab = pl.pallas_call(
        _gtab_body,
        grid=(_GBLK,),
        in_specs=[
            pl.BlockSpec((_GROWS, _D), lambda i: (jnp.minimum(i, _GBLK - 2), 0)),
            pl.BlockSpec((_D, _D), lambda i: (0, 0)),
            pl.BlockSpec((_D, _D), lambda i: (0, 0)),
            pl.BlockSpec((8, _D), lambda i: (0, 0)),
            pl.BlockSpec((1, _D), lambda i: (0, 0)),
        ],
        out_specs=pl.BlockSpec((_GROWS, _D), lambda i: (i, 0)),
        out_shape=jax.ShapeDtypeStruct((_GEXT, _D), f32),
    )(emb_table, wfc2.astype(jnp.bfloat16), W_lin.astype(jnp.bfloat16),
      nprj, bfc2d)

    cat3 = category.reshape(_B, 3, 96).astype(i32)
    tokp = jnp.concatenate([
        jnp.pad(brand.astype(i32), ((0, 0), (0, 4)), constant_values=1),
        name.astype(i32),
        cat3.reshape(_B, 288),
    ], axis=1)
    tok1 = (tokp == 1).astype(f32)
    cnt1 = jnp.stack([
        tok1[:, 0:16].sum(1), tok1[:, 16:48].sum(1), tok1[:, 48:144].sum(1),
        tok1[:, 144:240].sum(1), tok1[:, 240:336].sum(1)], axis=1)
    init = (-cnt1.reshape(-1)[:, None]) * emb_table[1][None, :]

    mesh = plsc.VectorSubcoreMesh(core_axis_name="c", subcore_axis_name="s")
    segsum = pl.kernel(
        _seg_body,
        mesh=mesh,
        compiler_params=pltpu.CompilerParams(use_tc_tiling_on_sc=False),
        out_type=jax.ShapeDtypeStruct((_B * 5, _D), f32),
        scratch_types=[
            pltpu.VMEM((_TPB,), i32),
            pltpu.VMEM((48, _D), f32),
            pltpu.VMEM((48, _D), f32),
            pltpu.VMEM((5, _D), f32),
            pltpu.SemaphoreType.DMA,
            pltpu.SemaphoreType.DMA,
        ],
    )(emb_table, tokp.reshape(-1), init)

    cmask = category != 1
    any1 = jnp.any(cmask, axis=-1)
    n1 = any1.sum(-1).astype(f32)
    n2 = jnp.any(any1, axis=-1).sum(-1).astype(f32)
    w3 = 1.0 / (n1 + 1e-6) / (n2 + 1e-6)[:, None]
    wpad = jnp.pad(w3, ((0, 0), (0, 125)))

    ppad = jnp.pad(price.astype(f32), ((0, 0), (0, 128 - 11)))
    rpad = jnp.pad(rating.astype(f32), ((0, 0), (0, 128 - 4)))
    wppad = jnp.pad(W_price, ((0, 128 - 11), (0, 0)))
    wrpad = jnp.pad(W_rating, ((0, 128 - 4), (0, 0)))
    bb = 128
    gext = pl.pallas_call(
        _small_body,
        grid=(_B // bb,),
        in_specs=[
            pl.BlockSpec((8, _D), lambda i: (0, 0)),
            pl.BlockSpec((bb, 128), lambda i: (i, 0)),
            pl.BlockSpec((bb, 128), lambda i: (i, 0)),
            pl.BlockSpec((128, _D), lambda i: (0, 0)),
            pl.BlockSpec((128, _D), lambda i: (0, 0)),
            pl.BlockSpec((bb, 5, _D), lambda i: (i, 0, 0)),
            pl.BlockSpec((bb, 128), lambda i: (i, 0)),
            pl.BlockSpec((8, _D), lambda i: (0, 0)),
            pl.BlockSpec((1, _D), lambda i: (0, 0)),
            pl.BlockSpec((_D, _D), lambda i: (0, 0)),
            pl.BlockSpec((_D, _D), lambda i: (0, 0)),
        ],
        out_specs=pl.BlockSpec((bb * 5, _D), lambda i: (_VPAD // (bb * 5) + i, 0)),
        out_shape=jax.ShapeDtypeStruct((_GEXT, _D), f32),
        input_output_aliases={0: 0},
    )(gtab, ppad, rpad, wppad, wrpad, segsum.reshape(_B, 5, _D), wpad,
      nprj, bfc2d, wfc2, W_lin)

    small_idx = (_VPAD + 5 * jnp.arange(_B, dtype=i32))[:, None] \
        + jnp.arange(5, dtype=i32)[None, :]
    ridx = jnp.concatenate([small_idx, description.astype(i32)],
                           axis=1).reshape(-1)
    out_flat = pl.kernel(
        _asm_body,
        mesh=mesh,
        out_type=jax.ShapeDtypeStruct((_B * 133, _D), f32),
        scratch_types=[
            pltpu.VMEM((_SPAN,), i32),
            pltpu.VMEM((_SPAN,), i32),
            pltpu.VMEM((_SPAN, _D), f32),
            pltpu.VMEM((_SPAN, _D), f32),
            pltpu.SemaphoreType.DMA,
            pltpu.SemaphoreType.DMA,
        ],
    )(gext, ridx)
    all_embeddings = out_flat.reshape(_B, 133, _D)
    all_masks = jnp.concatenate([
        price.sum(axis=1, keepdims=True) != 0.0,
        jnp.ones((_B, 1), bool),
        brand[:, :1] != 1,
        name[:, :1] != 1,
        jnp.ones((_B, 1), bool),
        description != 1,
    ], axis=1)
    return all_embeddings, all_masks


# speedup vs baseline: 3.4667x; 1.0158x over previous
import functools

import jax
import jax.numpy as jnp
from jax import lax
from jax.experimental import pallas as pl
from jax.experimental.pallas import tpu as pltpu
import jax.experimental.pallas.tpu_sc as plsc

_D = 1024
_V = 50265
_B = 1024

_NC = 2
_NS = 16
_NW = _NC * _NS
_BPW = _B // _NW
_SEGL = (16, 32, 96, 96, 96)
_SEGO = (0, 16, 48, 144, 240)
_TPB = 336

_GROWS = 512
_VPAD = 51200
_GBLK = _VPAD // _GROWS
_GEXT = _VPAD + _B * 5
_RPW = _B * 133 // _NW
_SPAN = 56


def _np_body(fnp_ref, wfc1_ref, out_ref):
    out_ref[...] = jnp.dot(fnp_ref[...], wfc1_ref[...],
                           preferred_element_type=jnp.float32)


def _gtab_body(x_ref, wfc2_ref, wlin_ref, np_ref, bfc_ref, out_ref):
    h = jnp.dot(x_ref[...].astype(jnp.bfloat16), wfc2_ref[...],
                preferred_element_type=jnp.float32)
    h = jnp.maximum(h + np_ref[5:6, :] + bfc_ref[...], 0.0)
    out_ref[...] = jnp.dot(h.astype(jnp.bfloat16), wlin_ref[...],
                           preferred_element_type=jnp.float32)


def _small_body(g_ref, pp_ref, rp_ref, wp_ref, wr_ref, s_ref, w_ref, np_ref,
                bfc_ref, wfc2_ref, wlin_ref, out_ref):
    del g_ref
    pv = jnp.dot(pp_ref[...], wp_ref[...], preferred_element_type=jnp.float32)
    rv = jnp.dot(rp_ref[...], wr_ref[...], preferred_element_type=jnp.float32)
    bs = s_ref[:, 0, :]
    ns = s_ref[:, 1, :]
    cv = (w_ref[:, 0:1] * s_ref[:, 2, :] + w_ref[:, 1:2] * s_ref[:, 3, :]
          + w_ref[:, 2:3] * s_ref[:, 4, :])
    v = jnp.stack((pv, rv, bs, ns, cv), axis=1).reshape(-1, _D)
    h = jnp.dot(v, wfc2_ref[...], preferred_element_type=jnp.float32)
    h = jnp.maximum(h + jnp.tile(np_ref[0:5, :], (pv.shape[0], 1))
                    + bfc_ref[...], 0.0)
    out_ref[...] = jnp.dot(h, wlin_ref[...], preferred_element_type=jnp.float32)


_CHUNKS = ((0, 0, 16), (1, 16, 32), (2, 48, 32), (2, 80, 32), (2, 112, 32),
           (3, 144, 32), (3, 176, 32), (3, 208, 32),
           (4, 240, 32), (4, 272, 32), (4, 304, 32))


def _seg_body(emb_hbm, tok_hbm, init_hbm, out_hbm,
              idx_v, bufa_v, bufb_v, bufc_v, acc_v, sem_a, sem_b, sem_c):
    wid = lax.axis_index("s") * _NC + lax.axis_index("c")
    bufs = (bufa_v, bufb_v, bufc_v)
    sems = (sem_a, sem_b, sem_c)

    def b_body(j, carry):
        b = wid * _BPW + j
        pltpu.sync_copy(tok_hbm.at[pl.ds(b * _TPB, _TPB)], idx_v)
        pltpu.sync_copy(init_hbm.at[pl.ds(b * 5, 5)], acc_v)
        cps = {}
        for i in (0, 1):
            s0, o0, n0 = _CHUNKS[i]
            cps[i] = pltpu.async_copy(emb_hbm.at[idx_v.at[pl.ds(o0, n0)]],
                                      bufs[i].at[pl.ds(0, n0)], sems[i])
        for i, (s, o, n) in enumerate(_CHUNKS):
            if i + 2 < len(_CHUNKS):
                s2, o2, n2 = _CHUNKS[i + 2]
                cps[i + 2] = pltpu.async_copy(
                    emb_hbm.at[idx_v.at[pl.ds(o2, n2)]],
                    bufs[(i + 2) % 3].at[pl.ds(0, n2)], sems[(i + 2) % 3])
            cps[i].wait()
            buf = bufs[i % 3]

            def cb_body(cb, c2, s=s, n=n, buf=buf):
                base = pl.multiple_of(cb * 256, 256)

                def r_body(r, accs):
                    t = tuple(accs[k] + buf[2 * r, pl.ds(base + k * 16, 16)]
                              for k in range(16))
                    return tuple(t[k] + buf[2 * r + 1,
                                            pl.ds(base + k * 16, 16)]
                                 for k in range(16))

                acc0 = tuple(acc_v[s, pl.ds(base + k * 16, 16)]
                             for k in range(16))
                accs = lax.fori_loop(0, n // 2, r_body, acc0)
                for k in range(16):
                    acc_v[s, pl.ds(base + k * 16, 16)] = accs[k]
                return c2

            lax.fori_loop(0, _D // 256, cb_body, 0)
        pltpu.sync_copy(acc_v, out_hbm.at[pl.ds(b * 5, 5)])
        return carry

    lax.fori_loop(0, _BPW, b_body, 0)


def _asm_body(g_hbm, ridx_hbm, out_hbm,
              idxa_v, idxb_v, rowsa_v, rowsb_v, sem_a, sem_b):
    wid = lax.axis_index("s") * _NC + lax.axis_index("c")
    base = wid * _RPW
    idxs = (idxa_v, idxb_v)
    rows = (rowsa_v, rowsb_v)
    sems = (sem_a, sem_b)
    nsp = _RPW // _SPAN

    pltpu.sync_copy(ridx_hbm.at[pl.ds(base, _SPAN)], idxa_v)
    pltpu.async_copy(g_hbm.at[idxa_v], rowsa_v, sem_a)

    def pair_body(p, carry):
        for k in range(2):
            i = p * 2 + k
            nk = (k + 1) % 2

            @pl.when(i + 1 < nsp)
            def _(i=i, nk=nk):
                pltpu.sync_copy(
                    ridx_hbm.at[pl.ds(base + (i + 1) * _SPAN, _SPAN)],
                    idxs[nk])
                pltpu.async_copy(g_hbm.at[idxs[nk]], rows[nk], sems[nk])

            pltpu.make_async_copy(g_hbm.at[idxs[k]], rows[k], sems[k]).wait()
            pltpu.sync_copy(rows[k], out_hbm.at[pl.ds(base + i * _SPAN,
                                                      _SPAN)])
        return carry

    lax.fori_loop(0, nsp // 2, pair_body, 0)


def kernel(field, price, rating, brand, name, category, description,
           emb_table, W_price, W_rating, W_fc, b_fc, W_lin):
    f32 = jnp.float32
    i32 = jnp.int32
    wfc1 = W_fc[:_D]
    wfc2 = W_fc[_D:]
    bfc2d = b_fc.reshape(1, _D)

    fnp = jnp.take(emb_table, field[:, 0], axis=0)
    fnp = jnp.concatenate([fnp, jnp.zeros((2, _D), f32)], axis=0)
    nprj = pl.pallas_call(
        _np_body,
        out_shape=jax.ShapeDtypeStruct((8, _D), f32),
    )(fnp, wfc1)

    gtab = pl.pallas_call(
        _gtab_body,
        grid=(_GBLK,),
        in_specs=[
            pl.BlockSpec((_GROWS, _D), lambda i: (jnp.minimum(i, _GBLK - 2), 0)),
            pl.BlockSpec((_D, _D), lambda i: (0, 0)),
            pl.BlockSpec((_D, _D), lambda i: (0, 0)),
            pl.BlockSpec((8, _D), lambda i: (0, 0)),
            pl.BlockSpec((1, _D), lambda i: (0, 0)),
        ],
        out_specs=pl.BlockSpec((_GROWS, _D), lambda i: (i, 0)),
        out_shape=jax.ShapeDtypeStruct((_GEXT, _D), f32),
    )(emb_table, wfc2.astype(jnp.bfloat16), W_lin.astype(jnp.bfloat16),
      nprj, bfc2d)

    cat3 = category.reshape(_B, 3, 96).astype(i32)
    tokp = jnp.concatenate([
        jnp.pad(brand.astype(i32), ((0, 0), (0, 4)), constant_values=1),
        name.astype(i32),
        cat3.reshape(_B, 288),
    ], axis=1)
    tok1 = (tokp == 1).astype(f32)
    cnt1 = jnp.stack([
        tok1[:, 0:16].sum(1), tok1[:, 16:48].sum(1), tok1[:, 48:144].sum(1),
        tok1[:, 144:240].sum(1), tok1[:, 240:336].sum(1)], axis=1)
    init = (-cnt1.reshape(-1)[:, None]) * emb_table[1][None, :]

    mesh = plsc.VectorSubcoreMesh(core_axis_name="c", subcore_axis_name="s")
    segsum = pl.kernel(
        _seg_body,
        mesh=mesh,
        compiler_params=pltpu.CompilerParams(use_tc_tiling_on_sc=False),
        out_type=jax.ShapeDtypeStruct((_B * 5, _D), f32),
        scratch_types=[
            pltpu.VMEM((_TPB,), i32),
            pltpu.VMEM((32, _D), f32),
            pltpu.VMEM((32, _D), f32),
            pltpu.VMEM((32, _D), f32),
            pltpu.VMEM((5, _D), f32),
            pltpu.SemaphoreType.DMA,
            pltpu.SemaphoreType.DMA,
            pltpu.SemaphoreType.DMA,
        ],
    )(emb_table, tokp.reshape(-1), init)

    cmask = category != 1
    any1 = jnp.any(cmask, axis=-1)
    n1 = any1.sum(-1).astype(f32)
    n2 = jnp.any(any1, axis=-1).sum(-1).astype(f32)
    w3 = 1.0 / (n1 + 1e-6) / (n2 + 1e-6)[:, None]
    wpad = jnp.pad(w3, ((0, 0), (0, 125)))

    ppad = jnp.pad(price.astype(f32), ((0, 0), (0, 128 - 11)))
    rpad = jnp.pad(rating.astype(f32), ((0, 0), (0, 128 - 4)))
    wppad = jnp.pad(W_price, ((0, 128 - 11), (0, 0)))
    wrpad = jnp.pad(W_rating, ((0, 128 - 4), (0, 0)))
    bb = 128
    gext = pl.pallas_call(
        _small_body,
        grid=(_B // bb,),
        in_specs=[
            pl.BlockSpec((8, _D), lambda i: (0, 0)),
            pl.BlockSpec((bb, 128), lambda i: (i, 0)),
            pl.BlockSpec((bb, 128), lambda i: (i, 0)),
            pl.BlockSpec((128, _D), lambda i: (0, 0)),
            pl.BlockSpec((128, _D), lambda i: (0, 0)),
            pl.BlockSpec((bb, 5, _D), lambda i: (i, 0, 0)),
            pl.BlockSpec((bb, 128), lambda i: (i, 0)),
            pl.BlockSpec((8, _D), lambda i: (0, 0)),
            pl.BlockSpec((1, _D), lambda i: (0, 0)),
            pl.BlockSpec((_D, _D), lambda i: (0, 0)),
            pl.BlockSpec((_D, _D), lambda i: (0, 0)),
        ],
        out_specs=pl.BlockSpec((bb * 5, _D), lambda i: (_VPAD // (bb * 5) + i, 0)),
        out_shape=jax.ShapeDtypeStruct((_GEXT, _D), f32),
        input_output_aliases={0: 0},
    )(gtab, ppad, rpad, wppad, wrpad, segsum.reshape(_B, 5, _D), wpad,
      nprj, bfc2d, wfc2, W_lin)

    small_idx = (_VPAD + 5 * jnp.arange(_B, dtype=i32))[:, None] \
        + jnp.arange(5, dtype=i32)[None, :]
    ridx = jnp.concatenate([small_idx, description.astype(i32)],
                           axis=1).reshape(-1)
    out_flat = pl.kernel(
        _asm_body,
        mesh=mesh,
        out_type=jax.ShapeDtypeStruct((_B * 133, _D), f32),
        scratch_types=[
            pltpu.VMEM((_SPAN,), i32),
            pltpu.VMEM((_SPAN,), i32),
            pltpu.VMEM((_SPAN, _D), f32),
            pltpu.VMEM((_SPAN, _D), f32),
            pltpu.SemaphoreType.DMA,
            pltpu.SemaphoreType.DMA,
        ],
    )(gext, ridx)
    all_embeddings = out_flat.reshape(_B, 133, _D)
    all_masks = jnp.concatenate([
        price.sum(axis=1, keepdims=True) != 0.0,
        jnp.ones((_B, 1), bool),
        brand[:, :1] != 1,
        name[:, :1] != 1,
        jnp.ones((_B, 1), bool),
        description != 1,
    ], axis=1)
    return all_embeddings, all_masks


# speedup vs baseline: 4.1903x; 1.2087x over previous
import functools

import jax
import jax.numpy as jnp
from jax import lax
from jax.experimental import pallas as pl
from jax.experimental.pallas import tpu as pltpu
import jax.experimental.pallas.tpu_sc as plsc

_D = 1024
_V = 50265
_B = 1024

_NC = 2
_NS = 16
_NW = _NC * _NS
_BPW = _B // _NW
_SEGL = (16, 32, 96, 96, 96)
_SEGO = (0, 16, 48, 144, 240)
_TPB = 336

_GROWS = 512
_VPAD = 51200
_GBLK = _VPAD // _GROWS
_GEXT = _VPAD + _B * 5
_RPW = _B * 136 // _NW
_SPAN = 32


def _np_body(fnp_ref, wfc1_ref, out_ref):
    out_ref[...] = jnp.dot(fnp_ref[...], wfc1_ref[...],
                           preferred_element_type=jnp.float32)


def _gtab_body(x_ref, wfc2_ref, wlin_ref, np_ref, bfc_ref, out_ref):
    h = jnp.dot(x_ref[...].astype(jnp.bfloat16), wfc2_ref[...],
                preferred_element_type=jnp.float32)
    h = jnp.maximum(h + np_ref[5:6, :] + bfc_ref[...], 0.0)
    out_ref[...] = jnp.dot(h.astype(jnp.bfloat16), wlin_ref[...],
                           preferred_element_type=jnp.float32)


def _small_body(g_ref, pp_ref, rp_ref, wp_ref, wr_ref, s_ref, w_ref, np_ref,
                bfc_ref, wfc2_ref, wlin_ref, out_ref):
    del g_ref
    pv = jnp.dot(pp_ref[...], wp_ref[...], preferred_element_type=jnp.float32)
    rv = jnp.dot(rp_ref[...], wr_ref[...], preferred_element_type=jnp.float32)
    bs = s_ref[:, 0, :]
    ns = s_ref[:, 1, :]
    cv = (w_ref[:, 0:1] * s_ref[:, 2, :] + w_ref[:, 1:2] * s_ref[:, 3, :]
          + w_ref[:, 2:3] * s_ref[:, 4, :])
    v = jnp.stack((pv, rv, bs, ns, cv), axis=1).reshape(-1, _D)
    h = jnp.dot(v, wfc2_ref[...], preferred_element_type=jnp.float32)
    h = jnp.maximum(h + jnp.tile(np_ref[0:5, :], (pv.shape[0], 1))
                    + bfc_ref[...], 0.0)
    out_ref[...] = jnp.dot(h, wlin_ref[...], preferred_element_type=jnp.float32)


_CHUNKS = ((0, 0, 16), (1, 16, 32), (2, 48, 32), (2, 80, 32), (2, 112, 32),
           (3, 144, 32), (3, 176, 32), (3, 208, 32),
           (4, 240, 32), (4, 272, 32), (4, 304, 32))


def _seg_body(emb_hbm, tok_hbm, init_hbm, out_hbm,
              idx_v, bufa_v, bufb_v, bufc_v, acc_v, sem_a, sem_b, sem_c):
    wid = lax.axis_index("s") * _NC + lax.axis_index("c")
    bufs = (bufa_v, bufb_v, bufc_v)
    sems = (sem_a, sem_b, sem_c)

    def b_body(j, carry):
        b = wid * _BPW + j
        pltpu.sync_copy(tok_hbm.at[pl.ds(b * _TPB, _TPB)], idx_v)
        pltpu.sync_copy(init_hbm.at[pl.ds(b * 5, 5)], acc_v)
        cps = {}
        for i in (0, 1):
            s0, o0, n0 = _CHUNKS[i]
            cps[i] = pltpu.async_copy(emb_hbm.at[idx_v.at[pl.ds(o0, n0)]],
                                      bufs[i].at[pl.ds(0, n0)], sems[i])
        for i, (s, o, n) in enumerate(_CHUNKS):
            if i + 2 < len(_CHUNKS):
                s2, o2, n2 = _CHUNKS[i + 2]
                cps[i + 2] = pltpu.async_copy(
                    emb_hbm.at[idx_v.at[pl.ds(o2, n2)]],
                    bufs[(i + 2) % 3].at[pl.ds(0, n2)], sems[(i + 2) % 3])
            cps[i].wait()
            buf = bufs[i % 3]

            def cb_body(cb, c2, s=s, n=n, buf=buf):
                base = pl.multiple_of(cb * 256, 256)

                def r_body(r, accs):
                    t = tuple(accs[k] + buf[2 * r, pl.ds(base + k * 16, 16)]
                              for k in range(16))
                    return tuple(t[k] + buf[2 * r + 1,
                                            pl.ds(base + k * 16, 16)]
                                 for k in range(16))

                acc0 = tuple(acc_v[s, pl.ds(base + k * 16, 16)]
                             for k in range(16))
                accs = lax.fori_loop(0, n // 2, r_body, acc0)
                for k in range(16):
                    acc_v[s, pl.ds(base + k * 16, 16)] = accs[k]
                return c2

            lax.fori_loop(0, _D // 256, cb_body, 0)
        pltpu.sync_copy(acc_v, out_hbm.at[pl.ds(b * 5, 5)])
        return carry

    lax.fori_loop(0, _BPW, b_body, 0)


def _asm_body(g_hbm, ridx_hbm, out_hbm,
              idxa_v, idxb_v, rowsa_v, rowsb_v, sem_a, sem_b):
    wid = lax.axis_index("s") * _NC + lax.axis_index("c")
    base = wid * _RPW
    idxs = (idxa_v, idxb_v)
    rows = (rowsa_v, rowsb_v)
    sems = (sem_a, sem_b)
    nsp = _RPW // _SPAN

    pltpu.sync_copy(ridx_hbm.at[pl.ds(base, _SPAN)], idxa_v)
    pltpu.async_copy(g_hbm.at[idxa_v], rowsa_v, sem_a)

    def pair_body(p, carry):
        for k in range(2):
            i = p * 2 + k
            nk = (k + 1) % 2

            @pl.when(i + 1 < nsp)
            def _(i=i, nk=nk):
                pltpu.sync_copy(
                    ridx_hbm.at[pl.ds(base + (i + 1) * _SPAN, _SPAN)],
                    idxs[nk])
                pltpu.async_copy(g_hbm.at[idxs[nk]], rows[nk], sems[nk])

            pltpu.make_async_copy(g_hbm.at[idxs[k]], rows[k], sems[k]).wait()
            pltpu.sync_copy(rows[k], out_hbm.at[pl.ds(base + i * _SPAN,
                                                      _SPAN)])
        return carry

    lax.fori_loop(0, nsp // 2, pair_body, 0)


def kernel(field, price, rating, brand, name, category, description,
           emb_table, W_price, W_rating, W_fc, b_fc, W_lin):
    f32 = jnp.float32
    i32 = jnp.int32
    wfc1 = W_fc[:_D]
    wfc2 = W_fc[_D:]
    bfc2d = b_fc.reshape(1, _D)

    fnp = jnp.take(emb_table, field[:, 0], axis=0)
    fnp = jnp.concatenate([fnp, jnp.zeros((2, _D), f32)], axis=0)
    nprj = pl.pallas_call(
        _np_body,
        out_shape=jax.ShapeDtypeStruct((8, _D), f32),
    )(fnp, wfc1)

    gtab = pl.pallas_call(
        _gtab_body,
        grid=(_GBLK,),
        in_specs=[
            pl.BlockSpec((_GROWS, _D), lambda i: (jnp.minimum(i, _GBLK - 2), 0)),
            pl.BlockSpec((_D, _D), lambda i: (0, 0)),
            pl.BlockSpec((_D, _D), lambda i: (0, 0)),
            pl.BlockSpec((8, _D), lambda i: (0, 0)),
            pl.BlockSpec((1, _D), lambda i: (0, 0)),
        ],
        out_specs=pl.BlockSpec((_GROWS, _D), lambda i: (i, 0)),
        out_shape=jax.ShapeDtypeStruct((_GEXT, _D), f32),
    )(emb_table, wfc2.astype(jnp.bfloat16), W_lin.astype(jnp.bfloat16),
      nprj, bfc2d)

    cat3 = category.reshape(_B, 3, 96).astype(i32)
    tokp = jnp.concatenate([
        jnp.pad(brand.astype(i32), ((0, 0), (0, 4)), constant_values=1),
        name.astype(i32),
        cat3.reshape(_B, 288),
    ], axis=1)
    tok1 = (tokp == 1).astype(f32)
    cnt1 = jnp.stack([
        tok1[:, 0:16].sum(1), tok1[:, 16:48].sum(1), tok1[:, 48:144].sum(1),
        tok1[:, 144:240].sum(1), tok1[:, 240:336].sum(1)], axis=1)
    init = (-cnt1.reshape(-1)[:, None]) * emb_table[1][None, :]

    mesh = plsc.VectorSubcoreMesh(core_axis_name="c", subcore_axis_name="s")
    segsum = pl.kernel(
        _seg_body,
        mesh=mesh,
        compiler_params=pltpu.CompilerParams(use_tc_tiling_on_sc=False),
        out_type=jax.ShapeDtypeStruct((_B * 5, _D), f32),
        scratch_types=[
            pltpu.VMEM((_TPB,), i32),
            pltpu.VMEM((32, _D), f32),
            pltpu.VMEM((32, _D), f32),
            pltpu.VMEM((32, _D), f32),
            pltpu.VMEM((5, _D), f32),
            pltpu.SemaphoreType.DMA,
            pltpu.SemaphoreType.DMA,
            pltpu.SemaphoreType.DMA,
        ],
    )(emb_table, tokp.reshape(-1), init)

    cmask = category != 1
    any1 = jnp.any(cmask, axis=-1)
    n1 = any1.sum(-1).astype(f32)
    n2 = jnp.any(any1, axis=-1).sum(-1).astype(f32)
    w3 = 1.0 / (n1 + 1e-6) / (n2 + 1e-6)[:, None]
    wpad = jnp.pad(w3, ((0, 0), (0, 125)))

    ppad = jnp.pad(price.astype(f32), ((0, 0), (0, 128 - 11)))
    rpad = jnp.pad(rating.astype(f32), ((0, 0), (0, 128 - 4)))
    wppad = jnp.pad(W_price, ((0, 128 - 11), (0, 0)))
    wrpad = jnp.pad(W_rating, ((0, 128 - 4), (0, 0)))
    bb = 128
    gext = pl.pallas_call(
        _small_body,
        grid=(_B // bb,),
        in_specs=[
            pl.BlockSpec((8, _D), lambda i: (0, 0)),
            pl.BlockSpec((bb, 128), lambda i: (i, 0)),
            pl.BlockSpec((bb, 128), lambda i: (i, 0)),
            pl.BlockSpec((128, _D), lambda i: (0, 0)),
            pl.BlockSpec((128, _D), lambda i: (0, 0)),
            pl.BlockSpec((bb, 5, _D), lambda i: (i, 0, 0)),
            pl.BlockSpec((bb, 128), lambda i: (i, 0)),
            pl.BlockSpec((8, _D), lambda i: (0, 0)),
            pl.BlockSpec((1, _D), lambda i: (0, 0)),
            pl.BlockSpec((_D, _D), lambda i: (0, 0)),
            pl.BlockSpec((_D, _D), lambda i: (0, 0)),
        ],
        out_specs=pl.BlockSpec((bb * 5, _D), lambda i: (_VPAD // (bb * 5) + i, 0)),
        out_shape=jax.ShapeDtypeStruct((_GEXT, _D), f32),
        input_output_aliases={0: 0},
    )(gtab, ppad, rpad, wppad, wrpad, segsum.reshape(_B, 5, _D), wpad,
      nprj, bfc2d, wfc2, W_lin)

    small_idx = (_VPAD + 5 * jnp.arange(_B, dtype=i32))[:, None] \
        + jnp.arange(5, dtype=i32)[None, :]
    ridx = jnp.concatenate([small_idx, description.astype(i32),
                            jnp.zeros((_B, 3), i32)],
                           axis=1).reshape(-1)
    out_flat = pl.kernel(
        _asm_body,
        mesh=mesh,
        out_type=jax.ShapeDtypeStruct((_B * 136, _D), f32),
        scratch_types=[
            pltpu.VMEM((_SPAN,), i32),
            pltpu.VMEM((_SPAN,), i32),
            pltpu.VMEM((_SPAN, _D), f32),
            pltpu.VMEM((_SPAN, _D), f32),
            pltpu.SemaphoreType.DMA,
            pltpu.SemaphoreType.DMA,
        ],
    )(gext, ridx)
    all_embeddings = out_flat.reshape(_B, 136, _D)[:, :133, :]
    all_masks = jnp.concatenate([
        price.sum(axis=1, keepdims=True) != 0.0,
        jnp.ones((_B, 1), bool),
        brand[:, :1] != 1,
        name[:, :1] != 1,
        jnp.ones((_B, 1), bool),
        description != 1,
    ], axis=1)
    return all_embeddings, all_masks
